# baseline (speedup 1.0000x reference)
import jax
import jax.numpy as jnp
from jax import lax
from jax.experimental import pallas as pl
from jax.experimental.pallas import tpu as pltpu

_S = 30.0


def _loss_sum(x, t, mrow):
    br, c = x.shape
    ones = jnp.ones((c, 1), jnp.float32)

    def msum(v):
        return jnp.dot(v, ones, preferred_element_type=jnp.float32)[:, 0]

    col = lax.broadcasted_iota(jnp.int32, (br, c), 1)
    tmask = col == t[:, None]
    p = msum(jnp.where(tmask, x, 0.0))
    bm = msum(jnp.where(tmask, mrow[None, :], 0.0))

    rmax = jnp.max(x, axis=1)
    expd = jnp.exp(_S * x - (_S * rmax)[:, None])
    e = msum(expd)
    t1 = msum(jnp.where(tmask, expd, 0.0))
    z = e - t1 + jnp.exp(_S * (p - bm - rmax))
    lossb = _S * rmax + jnp.log(z) - _S * (p - bm)
    return jnp.sum(lossb)


def _body(x_hbm, m_ref, t_ref, out_ref, b0, b1, s0, s1):
    i = pl.program_id(0)
    nb = pl.num_programs(0)
    br = b0.shape[0]

    def cp(j, buf, sem):
        return pltpu.make_async_copy(x_hbm.at[pl.ds(j * br, br), :], buf, sem)

    @pl.when(i == 0)
    def _p0():
        cp(0, b0, s0).start()

    nxt = i + 1

    @pl.when((nxt < nb) & (nxt % 2 == 0))
    def _p1():
        cp(nxt, b0, s0).start()

    @pl.when((nxt < nb) & (nxt % 2 == 1))
    def _p2():
        cp(nxt, b1, s1).start()

    @pl.when(i % 2 == 0)
    def _w0():
        cp(i, b0, s0).wait()

    @pl.when(i % 2 == 1)
    def _w1():
        cp(i, b1, s1).wait()

    use0 = (i % 2 == 0)
    x = jnp.where(use0, b0[...], b1[...])
    s = _loss_sum(x, t_ref[0, 0, :], m_ref[0, :])
    part = (s * (1.0 / (br * nb)))[None, None]

    @pl.when(i == 0)
    def _init():
        out_ref[...] = jnp.zeros((1, 1), jnp.float32)

    out_ref[...] += part


def kernel(x, m_list, target):
    b, c = x.shape
    br = 2048
    nb = b // br
    t3 = target.astype(jnp.int32).reshape(nb, 1, br)
    m2 = m_list.reshape(1, c)
    out = pl.pallas_call(
        _body,
        grid=(nb,),
        in_specs=[
            pl.BlockSpec(memory_space=pltpu.MemorySpace.HBM),
            pl.BlockSpec((1, c), lambda i: (0, 0)),
            pl.BlockSpec((1, 1, br), lambda i: (i, 0, 0)),
        ],
        out_specs=pl.BlockSpec((1, 1), lambda i: (0, 0)),
        out_shape=jax.ShapeDtypeStruct((1, 1), jnp.float32),
        scratch_shapes=[
            pltpu.VMEM((br, c), jnp.float32),
            pltpu.VMEM((br, c), jnp.float32),
            pltpu.SemaphoreType.DMA,
            pltpu.SemaphoreType.DMA,
        ],
    )(x, m2, t3)
    return out[0, 0]
